# Initial kernel scaffold; baseline (speedup 1.0000x reference)
import jax
import jax.numpy as jnp
from jax.experimental import pallas as pl


def kernel(X, adj, W_in, b_in, g0_W, g0_att_src, g0_att_dst, g0_b, g1_W, g1_att_src, g1_att_dst, g1_b, W_mlp, b_mlp):
    raise NotImplementedError("write your pallas kernel here")


# speedup vs baseline: 5174.3909x; 5174.3909x over previous
import jax
import jax.numpy as jnp
from jax.experimental import pallas as pl


def _leaky_relu(x):
    return jnp.where(x >= 0, x, 0.2 * x)


def _elu(x):
    return jnp.where(x > 0, x, jnp.exp(jnp.minimum(x, 0.0)) - 1.0)


def _gat_model_kernel(adjt_ref, X_ref, W_in_ref, b_in_ref,
                      g0_W_ref, g0_as_ref, g0_ad_ref, g0_b_ref,
                      g1_W_ref, g1_as_ref, g1_ad_ref, g1_b_ref,
                      W_mlp_ref, b_mlp_ref, out_ref):
    N = adjt_ref.shape[0]
    adjt = adjt_ref[...]
    row = jax.lax.broadcasted_iota(jnp.int32, (N, N), 0)
    col = jax.lax.broadcasted_iota(jnp.int32, (N, N), 1)
    eye = row == col
    maskt = adjt > 0.0
    validt = jnp.logical_or(maskt, eye)
    countt = maskt.astype(jnp.float32) + eye.astype(jnp.float32)

    x = jnp.dot(X_ref[...], W_in_ref[...],
                preferred_element_type=jnp.float32) + b_in_ref[...]

    def gat(x, W, a_src, a_dst, b):
        h = jnp.dot(x, W, preferred_element_type=jnp.float32)
        a_s = jnp.sum(h * a_src, axis=1)
        a_d = jnp.sum(h * a_dst, axis=1)
        e = _leaky_relu(a_d[:, None] + a_s[None, :])
        m = jnp.max(jnp.where(validt, e, -1e30), axis=1)
        p = countt * jnp.exp(jnp.minimum(e - m[:, None], 0.0))
        denom = jnp.sum(p, axis=1)
        w = p / (denom + 1e-16)[:, None]
        return jnp.dot(w, h, preferred_element_type=jnp.float32) + b

    x = _elu(gat(x, g0_W_ref[...], g0_as_ref[...], g0_ad_ref[...],
                 g0_b_ref[...]))
    x = _elu(gat(x, g1_W_ref[...], g1_as_ref[...], g1_ad_ref[...],
                 g1_b_ref[...]))
    o = jnp.dot(x, W_mlp_ref[...],
                preferred_element_type=jnp.float32) + b_mlp_ref[...]
    o = jnp.exp(o - jnp.max(o, axis=1, keepdims=True))
    out_ref[...] = o / jnp.sum(o, axis=1, keepdims=True)


def kernel(X, adj, W_in, b_in, g0_W, g0_att_src, g0_att_dst, g0_b,
           g1_W, g1_att_src, g1_att_dst, g1_b, W_mlp, b_mlp):
    N = X.shape[0]
    D_out = W_mlp.shape[1]
    v = lambda a: a.reshape(1, -1)
    return pl.pallas_call(
        _gat_model_kernel,
        out_shape=jax.ShapeDtypeStruct((N, D_out), jnp.float32),
    )(adj.T, X, W_in, v(b_in),
      g0_W, v(g0_att_src), v(g0_att_dst), v(g0_b),
      g1_W, v(g1_att_src), v(g1_att_dst), v(g1_b),
      W_mlp, v(b_mlp))


# speedup vs baseline: 7366.7943x; 1.4237x over previous
import jax
import jax.numpy as jnp
from jax.experimental import pallas as pl


def _leaky_relu(x):
    return jnp.where(x >= 0, x, 0.2 * x)


def _elu(x):
    return jnp.where(x > 0, x, jnp.exp(jnp.minimum(x, 0.0)) - 1.0)


def _gat_model_kernel(adj_ref, X_ref, W_in_ref, b_in_ref,
                      g0_W_ref, g0_as_ref, g0_ad_ref, g0_b_ref,
                      g1_W_ref, g1_as_ref, g1_ad_ref, g1_b_ref,
                      W_mlp_ref, b_mlp_ref, out_ref):
    mask = adj_ref[...] > 0.0

    x = jnp.dot(X_ref[...], W_in_ref[...],
                preferred_element_type=jnp.float32) + b_in_ref[...]

    def gat(x, W, a_src, a_dst, b):
        h = jnp.dot(x, W, preferred_element_type=jnp.float32)
        a_s = jnp.sum(h * a_src, axis=1)
        a_d = jnp.sum(h * a_dst, axis=1)
        e = _leaky_relu(a_s[:, None] + a_d[None, :])
        e_diag = _leaky_relu(a_s + a_d)
        m = jnp.maximum(jnp.max(jnp.where(mask, e, -1e30), axis=0), e_diag)
        pe = jnp.exp(jnp.minimum(e - m[None, :], 0.0))
        q = jnp.where(mask, pe, 0.0)
        d = jnp.exp(jnp.minimum(e_diag - m, 0.0))
        denom = jnp.sum(q, axis=0) + d
        agg = jax.lax.dot_general(q, h, (((0,), (0,)), ((), ())),
                                  preferred_element_type=jnp.float32)
        agg = agg + d[:, None] * h
        return agg / (denom + 1e-16)[:, None] + b

    x = _elu(gat(x, g0_W_ref[...], g0_as_ref[...], g0_ad_ref[...],
                 g0_b_ref[...]))
    x = _elu(gat(x, g1_W_ref[...], g1_as_ref[...], g1_ad_ref[...],
                 g1_b_ref[...]))
    o = jnp.dot(x, W_mlp_ref[...],
                preferred_element_type=jnp.float32) + b_mlp_ref[...]
    o = jnp.exp(o - jnp.max(o, axis=1, keepdims=True))
    out_ref[...] = o / jnp.sum(o, axis=1, keepdims=True)


def kernel(X, adj, W_in, b_in, g0_W, g0_att_src, g0_att_dst, g0_b,
           g1_W, g1_att_src, g1_att_dst, g1_b, W_mlp, b_mlp):
    N = X.shape[0]
    D_out = W_mlp.shape[1]
    v = lambda a: a.reshape(1, -1)
    return pl.pallas_call(
        _gat_model_kernel,
        out_shape=jax.ShapeDtypeStruct((N, D_out), jnp.float32),
    )(adj, X, W_in, v(b_in),
      g0_W, v(g0_att_src), v(g0_att_dst), v(g0_b),
      g1_W, v(g1_att_src), v(g1_att_dst), v(g1_b),
      W_mlp, v(b_mlp))
